# Initial kernel scaffold; baseline (speedup 1.0000x reference)
import jax
import jax.numpy as jnp
from jax.experimental import pallas as pl


def kernel(input_feats, edge_index, W1, b1, a1, W2, b2, a2):
    raise NotImplementedError("write your pallas kernel here")


# speedup vs baseline: 8.0045x; 8.0045x over previous
import functools

import jax
import jax.numpy as jnp
from jax import lax
from jax.experimental import pallas as pl
from jax.experimental.pallas import tpu as pltpu
from jax.experimental.pallas import tpu_sc as plsc

N = 10000
E = 160000
D = 256
DH = 128
NSC = 2
NT = 16
BATCH = 128
RING = 16
NGRP = 5
NBATCH = RING * NGRP
EPT = E // NT
EPT_PAD = NBATCH * BATCH
NPAD = EPT_PAD
ZCH = NPAD // NT // BATCH

RB = 1000
NRB = N // RB
DCH = 1000
DG = 100


def _mm_body(x_ref, w_ref, o_ref):
    o_ref[...] = jnp.dot(x_ref[...], w_ref[...],
                         preferred_element_type=jnp.float32)


def _matmul_halves(x, w):
    return pl.pallas_call(
        _mm_body,
        grid=(NRB, 2),
        in_specs=[
            pl.BlockSpec((RB, D), lambda i, j: (i, 0)),
            pl.BlockSpec((D, DH), lambda i, j: (0, j)),
        ],
        out_specs=pl.BlockSpec((RB, DH), lambda i, j: (i + NRB * j, 0)),
        out_shape=jax.ShapeDtypeStruct((NSC * N, DH), jnp.float32),
    )(x, w)


def _deg_body(dst_ref, o_ref):
    i = pl.program_id(0)
    dstv = dst_ref[0, 0, :]
    hi = (dstv // DG)[:, None]
    lo = (dstv % DG)[:, None]
    cols = lax.broadcasted_iota(jnp.int32, (DCH, DG), 1)
    oh_hi = (hi == cols).astype(jnp.float32)
    oh_lo = (lo == cols).astype(jnp.float32)
    part = lax.dot_general(oh_hi, oh_lo, (((0,), (0,)), ((), ())),
                           preferred_element_type=jnp.float32)

    @pl.when(i == 0)
    def _():
        o_ref[...] = part

    @pl.when(i > 0)
    def _():
        o_ref[...] += part


def _degree(dst):
    dst3 = dst.reshape(E // DCH, 1, DCH)
    degm = pl.pallas_call(
        _deg_body,
        grid=(E // DCH,),
        in_specs=[pl.BlockSpec((1, 1, DCH), lambda i: (i, 0, 0))],
        out_specs=pl.BlockSpec((DG, DG), lambda i: (0, 0)),
        out_shape=jax.ShapeDtypeStruct((DG, DG), jnp.float32),
    )(dst3)
    return degm.reshape(N, 1)


def _mid_body(slo, shi, glo, ghi, dref, b, a, w_ref, o_ref):
    denom = dref[...] + 1.0
    tlo = (slo[...] + glo[...]) / denom + b[:, :DH]
    thi = (shi[...] + ghi[...]) / denom + b[:, DH:]
    h = jnp.concatenate([tlo, thi], axis=1)
    h = jnp.where(h >= 0, h, a[0, 0] * h)
    o_ref[...] = jnp.dot(h, w_ref[...], preferred_element_type=jnp.float32)


def _mid_layer(s_lo, s_hi, g, d, b, a, w):
    lo = lambda i, j: (i, 0)
    hi = lambda i, j: (i + NRB, 0)
    return pl.pallas_call(
        _mid_body,
        grid=(NRB, 2),
        in_specs=[
            pl.BlockSpec((RB, DH), lo), pl.BlockSpec((RB, DH), lo),
            pl.BlockSpec((RB, DH), lo), pl.BlockSpec((RB, DH), hi),
            pl.BlockSpec((RB, 1), lo),
            pl.BlockSpec((1, D), lambda i, j: (0, 0)),
            pl.BlockSpec((1, 1), lambda i, j: (0, 0)),
            pl.BlockSpec((D, DH), lambda i, j: (0, j)),
        ],
        out_specs=pl.BlockSpec((RB, DH), lambda i, j: (i + NRB * j, 0)),
        out_shape=jax.ShapeDtypeStruct((NSC * N, DH), jnp.float32),
    )(s_lo, s_hi, g, g, d, b, a, w)


def _fin_body(slo, shi, glo, ghi, dref, b, a, o_ref):
    denom = dref[...] + 1.0
    tlo = (slo[...] + glo[...]) / denom + b[:, :DH]
    thi = (shi[...] + ghi[...]) / denom + b[:, DH:]
    t = jnp.concatenate([tlo, thi], axis=1)
    o_ref[...] = jnp.where(t >= 0, t, a[0, 0] * t)


def _fin_layer(s_lo, s_hi, g, d, b, a):
    lo = lambda i: (i, 0)
    hi = lambda i: (i + NRB, 0)
    return pl.pallas_call(
        _fin_body,
        grid=(NRB,),
        in_specs=[
            pl.BlockSpec((RB, DH), lo), pl.BlockSpec((RB, DH), lo),
            pl.BlockSpec((RB, DH), lo), pl.BlockSpec((RB, DH), hi),
            pl.BlockSpec((RB, 1), lo),
            pl.BlockSpec((1, D), lambda i: (0, 0)),
            pl.BlockSpec((1, 1), lambda i: (0, 0)),
        ],
        out_specs=pl.BlockSpec((RB, D), lambda i: (i, 0)),
        out_shape=jax.ShapeDtypeStruct((N, D), jnp.float32),
    )(s_lo, s_hi, g, g, d, b, a)


@functools.lru_cache(maxsize=None)
def _build_sc_agg():
    mesh = plsc.VectorSubcoreMesh(core_axis_name="c", subcore_axis_name="s",
                                  num_cores=NSC, num_subcores=NT)
    out_type = [jax.ShapeDtypeStruct((NPAD, DH), jnp.float32),
                jax.ShapeDtypeStruct((NPAD, DH), jnp.float32)]
    scratch = [
        pltpu.VMEM((RING, BATCH), jnp.int32),
        pltpu.VMEM((RING, BATCH), jnp.int32),
        pltpu.VMEM((BATCH, DH), jnp.float32),
        pltpu.VMEM((BATCH, DH), jnp.float32),
        pltpu.VMEM_SHARED((NPAD, DH), jnp.float32),
        pltpu.SemaphoreType.DMA,
        pltpu.SemaphoreType.DMA,
    ]

    def body(g_hbm, src_hbm, dst_hbm, z_hbm, out_lo, out_hi,
             src_v, dst_v, buf0, buf1, agg_sh, sem0, sem1):
        c = lax.axis_index("c")
        s = lax.axis_index("s")
        w = c * NT + s

        pltpu.sync_copy(z_hbm, buf0)
        for i in range(ZCH):
            pltpu.sync_copy(buf0,
                            agg_sh.at[pl.ds((s * ZCH + i) * BATCH, BATCH)])
        plsc.subcore_barrier()

        def group(gi, _):
            pltpu.sync_copy(src_hbm.at[w * NGRP + gi], src_v)
            pltpu.sync_copy(dst_hbm.at[s * NGRP + gi], dst_v)
            pltpu.async_copy(g_hbm.at[src_v.at[0]], buf0, sem0)

            def pair(k, _):
                j0 = 2 * k
                pltpu.async_copy(g_hbm.at[src_v.at[j0 + 1]], buf1, sem1)
                pltpu.make_async_copy(g_hbm.at[src_v.at[j0]], buf0,
                                      sem0).wait()
                pltpu.sync_copy(buf0, agg_sh.at[dst_v.at[j0]], add=True)

                @pl.when(k < RING // 2 - 1)
                def _():
                    pltpu.async_copy(g_hbm.at[src_v.at[j0 + 2]], buf0, sem0)

                pltpu.make_async_copy(g_hbm.at[src_v.at[j0 + 1]], buf1,
                                      sem1).wait()
                pltpu.sync_copy(buf1, agg_sh.at[dst_v.at[j0 + 1]], add=True)
                return 0

            lax.fori_loop(0, RING // 2, pair, 0)
            return 0

        lax.fori_loop(0, NGRP, group, 0)
        plsc.subcore_barrier()

        for i in range(ZCH):
            r0 = (s * ZCH + i) * BATCH
            pltpu.sync_copy(agg_sh.at[pl.ds(r0, BATCH)], buf0)

            @pl.when(c == 0)
            def _():
                pltpu.sync_copy(buf0, out_lo.at[pl.ds(r0, BATCH)])

            @pl.when(c == 1)
            def _():
                pltpu.sync_copy(buf0, out_hi.at[pl.ds(r0, BATCH)])

    return pl.kernel(body, out_type=out_type, mesh=mesh,
                     scratch_types=scratch)


def kernel(input_feats, edge_index, W1, b1, a1, W2, b2, a2):
    src = edge_index[0]
    dst = edge_index[1]

    pad = EPT_PAD - EPT
    pad_src = jnp.broadcast_to((jnp.arange(pad, dtype=jnp.int32) * 37) % N,
                               (NT, pad))
    pad_dst = jnp.broadcast_to(N + jnp.arange(pad, dtype=jnp.int32),
                               (NT, pad))
    srcr = jnp.concatenate([src.reshape(NT, EPT), pad_src], axis=1)
    dstr = jnp.concatenate([dst.reshape(NT, EPT), pad_dst], axis=1)
    srcr = srcr.reshape(NT, NBATCH, BATCH)
    src2 = jnp.concatenate([srcr, srcr + N]).reshape(NSC * NT * NGRP,
                                                     RING, BATCH)
    dst3 = dstr.reshape(NT * NGRP, RING, BATCH)

    zeros = jnp.zeros((BATCH, DH), jnp.float32)
    b1r = b1.reshape(1, D)
    b2r = b2.reshape(1, D)
    a1r = a1.reshape(1, 1)
    a2r = a2.reshape(1, 1)

    d = _degree(dst)
    sc_agg = _build_sc_agg()
    g1 = _matmul_halves(input_feats, W1)
    s1_lo, s1_hi = sc_agg(g1, src2, dst3, zeros)
    g2 = _mid_layer(s1_lo, s1_hi, g1, d, b1r, a1r, W2)
    s2_lo, s2_hi = sc_agg(g2, src2, dst3, zeros)
    return _fin_layer(s2_lo, s2_hi, g2, d, b2r, a2r)
